# baseline (speedup 1.0000x reference)
import functools

import numpy as np
import jax
import jax.numpy as jnp
from jax import lax
from jax.experimental import pallas as pl
from jax.experimental.pallas import tpu as pltpu
from jax.sharding import Mesh, PartitionSpec as P

try:
    from jax import shard_map as _shard_map
except ImportError:
    from jax.experimental.shard_map import shard_map as _shard_map

_BATCH = 4096
_NCARDS = 26
_CARD = 100
_WIDTH = _NCARDS * _CARD
_BR = 512


def _onehot_block(x_ref, r_ref, o_ref):
    xr = jnp.dot(x_ref[...], r_ref[...], preferred_element_type=jnp.float32)
    j = lax.broadcasted_iota(jnp.int32, o_ref.shape, 1)
    pos = (j - (j // _CARD) * _CARD).astype(jnp.float32)
    o_ref[...] = (xr == pos).astype(jnp.int32)


def _onehot_call(xb, rep, rows):
    return pl.pallas_call(
        _onehot_block,
        grid=(rows // _BR,),
        in_specs=[
            pl.BlockSpec((_BR, _NCARDS), lambda r: (r, 0)),
            pl.BlockSpec((_NCARDS, _WIDTH), lambda r: (0, 0)),
        ],
        out_specs=pl.BlockSpec((_BR, _WIDTH), lambda r: (r, 0)),
        out_shape=jax.ShapeDtypeStruct((rows, _WIDTH), jnp.int32),
        compiler_params=pltpu.CompilerParams(
            dimension_semantics=("parallel",)
        ),
    )(xb, rep)


def kernel(x):
    xb = x.astype(jnp.bfloat16)
    card_of_col = jnp.arange(_WIDTH, dtype=jnp.int32) // _CARD
    rep = (card_of_col[None, :] == jnp.arange(_NCARDS, dtype=jnp.int32)[:, None]
           ).astype(jnp.bfloat16)
    devs = jax.devices()
    if len(devs) >= 2:
        mesh = Mesh(np.array(devs[:2]), ("b",))
        f = _shard_map(
            functools.partial(_onehot_call, rows=_BATCH // 2),
            mesh=mesh,
            in_specs=(P("b", None), P(None, None)),
            out_specs=P("b", None),
            check_vma=False,
        )
        return f(xb, rep)
    return _onehot_call(xb, rep, _BATCH)


# speedup vs baseline: 5.7634x; 5.7634x over previous
import dataclasses

import jax
import jax.numpy as jnp
from jax import lax
from jax.experimental import pallas as pl
from jax.experimental.pallas import tpu as pltpu
from jax.experimental.pallas import tpu_sc as plsc

_BATCH = 4096
_NCARDS = 26
_CARD = 100
_WIDTH = _NCARDS * _CARD
_NC, _NS = 2, 16
_NW = _NC * _NS
_ROWS_W = _BATCH // _NW
_BLK = 16
_NBLK = _ROWS_W // _BLK
_IDX_BLK = _BLK * _NCARDS
_NVEC = _IDX_BLK // 16


def _scatter_block(buf, xbuf, val):
    for v in range(_NVEC):
        p = v * 16 + lax.iota(jnp.int32, 16)
        xv = xbuf[pl.ds(v * 16, 16)]
        row = p // _NCARDS
        col = (p % _NCARDS) * _CARD + xv
        plsc.store_scatter(buf, [row, col], val)


def _sc_onehot(zeros_hbm, idx_hbm, out_hbm, buf, xbuf, sem):
    wid = lax.axis_index("s") * _NC + lax.axis_index("c")
    row0 = wid * _ROWS_W
    ones = jnp.full((16,), 1, jnp.int32)
    zeros = jnp.zeros((16,), jnp.int32)

    pltpu.async_copy(zeros_hbm, buf, sem).wait()
    pltpu.sync_copy(idx_hbm.at[pl.ds(row0 * _NCARDS, _IDX_BLK)], xbuf)

    @pl.loop(0, _NBLK)
    def _(blk):
        _scatter_block(buf, xbuf, zeros)
        pltpu.sync_copy(
            idx_hbm.at[pl.ds((row0 + blk * _BLK) * _NCARDS, _IDX_BLK)], xbuf)
        _scatter_block(buf, xbuf, ones)
        pltpu.sync_copy(buf, out_hbm.at[pl.ds(row0 + blk * _BLK, _BLK)])


def kernel(x):
    idx = x.reshape(_BATCH * _NCARDS)
    zeros2d = jnp.zeros((_BLK, _WIDTH), jnp.int32)
    mesh = plsc.VectorSubcoreMesh(core_axis_name="c", subcore_axis_name="s")
    cp = pltpu.CompilerParams()
    if "needs_layout_passes" in pltpu.CompilerParams.__dataclass_fields__:
        cp = dataclasses.replace(cp, needs_layout_passes=False)
    run = pl.kernel(
        _sc_onehot,
        out_type=jax.ShapeDtypeStruct((_BATCH, _WIDTH), jnp.int32),
        mesh=mesh,
        scratch_types=[
            pltpu.VMEM((_BLK, _WIDTH), jnp.int32),
            pltpu.VMEM((_IDX_BLK,), jnp.int32),
            pltpu.SemaphoreType.DMA,
        ],
        compiler_params=cp,
    )
    return run(zeros2d, idx)


# speedup vs baseline: 8.6803x; 1.5061x over previous
import jax
import jax.numpy as jnp
from jax import lax
from jax.experimental import pallas as pl
from jax.experimental.pallas import tpu as pltpu

_BATCH = 4096
_NCARDS = 26
_CARD = 100
_WIDTH = _NCARDS * _CARD
_ALIGNED = 2560
_BR = 512
_NBLK = _BATCH // _BR


def _copies(scratch, o_ref, buf, blk, sem_a, sem_b):
    rows = pl.ds(blk * _BR, _BR)
    main = pltpu.make_async_copy(
        scratch.at[buf, :, pl.ds(0, _ALIGNED)],
        o_ref.at[rows, pl.ds(0, _ALIGNED)],
        sem_a.at[buf],
    )
    tail = pltpu.make_async_copy(
        scratch.at[buf, :, pl.ds(_ALIGNED, _WIDTH - _ALIGNED)],
        o_ref.at[rows, pl.ds(_ALIGNED, _WIDTH - _ALIGNED)],
        sem_b.at[buf],
    )
    return main, tail


def _onehot(x_ref, r_ref, o_ref, scratch, sem_a, sem_b):
    j = lax.broadcasted_iota(jnp.int32, (_BR, _WIDTH), 1)
    pos = (j - (j // _CARD) * _CARD).astype(jnp.float32)
    for blk in range(_NBLK):
        buf = blk % 2
        if blk >= 2:
            pm, pt = _copies(scratch, o_ref, buf, blk - 2, sem_a, sem_b)
            pm.wait()
            pt.wait()
        xr = jnp.dot(x_ref[pl.ds(blk * _BR, _BR), :], r_ref[...],
                     preferred_element_type=jnp.float32)
        scratch[buf] = (xr == pos).astype(jnp.int32)
        m, t = _copies(scratch, o_ref, buf, blk, sem_a, sem_b)
        m.start()
        t.start()
    for blk in range(_NBLK - 2, _NBLK):
        pm, pt = _copies(scratch, o_ref, blk % 2, blk, sem_a, sem_b)
        pm.wait()
        pt.wait()


def kernel(x):
    xb = x.astype(jnp.bfloat16)
    card_of_col = jnp.arange(_WIDTH, dtype=jnp.int32) // _CARD
    rep = (card_of_col[None, :] == jnp.arange(_NCARDS, dtype=jnp.int32)[:, None]
           ).astype(jnp.bfloat16)
    return pl.pallas_call(
        _onehot,
        in_specs=[
            pl.BlockSpec(memory_space=pltpu.VMEM),
            pl.BlockSpec(memory_space=pltpu.VMEM),
        ],
        out_specs=pl.BlockSpec(memory_space=pl.ANY),
        out_shape=jax.ShapeDtypeStruct((_BATCH, _WIDTH), jnp.int32),
        scratch_shapes=[
            pltpu.VMEM((2, _BR, _WIDTH), jnp.int32),
            pltpu.SemaphoreType.DMA((2,)),
            pltpu.SemaphoreType.DMA((2,)),
        ],
    )(xb, rep)
